# baseline (speedup 1.0000x reference)
import jax
import jax.numpy as jnp
from jax import lax
from jax.experimental import pallas as pl
from jax.experimental.pallas import tpu as pltpu
from jax.experimental.pallas import tpu_sc as plsc

N = 320000
D = 128
S = 3200
C = 8
NC = 2
NS = 16
NW = NC * NS
GROUP = 128
G = N // GROUP
SROWS = 3328
TRASH = S
ZROWS_PER_TILE = SROWS // NS
ROWS_PER_TILE = S // NS


def _sc_body(x_hbm, seg_hbm, y_hbm, psum_hbm, pyc_hbm,
             xbuf, segbuf, ybuf, fybuf, fidxv, aycbuf, colbuf, cidxbuf,
             crow, ssum, syc, lsem, ssem):
  cid = lax.axis_index("c")
  sid = lax.axis_index("s")
  wid = cid * NS + sid

  zeros16 = jnp.zeros((16,), jnp.float32)
  trash16 = jnp.full((16,), TRASH, jnp.int32)
  iota16 = lax.iota(jnp.int32, 16)

  def zrow(r, _):
    for c in range(D // 16):
      xbuf[0, r, pl.ds(c * 16, 16)] = zeros16
      fybuf[r, pl.ds(c * 16, 16)] = zeros16
    return 0
  lax.fori_loop(0, GROUP, zrow, 0)
  def zcol(r, _):
    for c in range(D // 16):
      colbuf[r, pl.ds(c * 16, 16)] = zeros16
    return 0
  lax.fori_loop(0, 16, zcol, 0)
  for b in range(8):
    fidxv[pl.ds(b * 16, 16)] = trash16

  zbase = sid * ZROWS_PER_TILE
  pltpu.sync_copy(xbuf.at[0, pl.ds(0, 128)], ssum.at[pl.ds(zbase, 128)])
  pltpu.sync_copy(xbuf.at[0, pl.ds(0, 80)], ssum.at[pl.ds(zbase + 128, 80)])
  pltpu.sync_copy(xbuf.at[0, pl.ds(0, 128)], syc.at[pl.ds(zbase, 128)])
  pltpu.sync_copy(xbuf.at[0, pl.ds(0, 80)], syc.at[pl.ds(zbase + 128, 80)])
  plsc.subcore_barrier()

  g_lo = wid * G // NW
  g_hi = (wid + 1) * G // NW

  def start_loads(g, p):
    pltpu.async_copy(x_hbm.at[pl.ds(g * GROUP, GROUP)], xbuf.at[p],
                     lsem.at[p])
    pltpu.async_copy(seg_hbm.at[g], segbuf.at[p], lsem.at[p])
    pltpu.async_copy(y_hbm.at[g], ybuf.at[p], lsem.at[p])

  def wait_loads(g, p):
    pltpu.make_async_copy(x_hbm.at[pl.ds(g * GROUP, GROUP)], xbuf.at[p],
                          lsem.at[p]).wait()
    pltpu.make_async_copy(seg_hbm.at[g], segbuf.at[p], lsem.at[p]).wait()
    pltpu.make_async_copy(y_hbm.at[g], ybuf.at[p], lsem.at[p]).wait()

  def start_xscatter(p):
    pltpu.async_copy(xbuf.at[p], ssum.at[segbuf.at[p]], ssem.at[p],
                     add=True)

  def wait_xscatter(p):
    pltpu.make_async_copy(xbuf.at[p], ssum.at[segbuf.at[p]],
                          ssem.at[p]).wait()

  def scatter_counts():
    pltpu.sync_copy(fybuf, syc.at[fidxv], add=True)
    for b in range(8):
      fidxv[pl.ds(b * 16, 16)] = trash16

  def flush(seg_id, fcount):
    fybuf[fcount, pl.ds(0, 16)] = aycbuf[pl.ds(0, 16)]
    blk = (fcount // 16) * 16
    off = fcount % 16
    v = fidxv[pl.ds(blk, 16)]
    fidxv[pl.ds(blk, 16)] = jnp.where(iota16 == off, seg_id, v)

  start_loads(g_lo, 0)

  def group_body(g, carry):
    cur_seg, fcount, valid, pend0, pend1 = carry
    p = (g - g_lo) % 2
    pend_q = jnp.where(p == 0, pend1, pend0)

    @pl.when(pend_q == 1)
    def _():
      wait_xscatter(1 - p)

    @pl.when(g + 1 < g_hi)
    def _():
      start_loads(g + 1, 1 - p)

    wait_loads(g, p)

    gseg_a = segbuf[p, pl.ds(0, 16)]
    gseg_b = segbuf[p, pl.ds(112, 16)]
    g_first = gseg_a[0]
    g_uniform = g_first == gseg_b[15]

    @pl.when(g_uniform)
    def _():
      def csum(k, accs):
        naccs = []
        for c in range(8):
          s = accs[c]
          for j in range(16):
            s = s + xbuf[p, k * 16 + j, pl.ds(c * 16, 16)]
          naccs.append(s)
        return tuple(naccs)
      tot = lax.fori_loop(0, GROUP // 16, csum,
                          tuple(zeros16 for _ in range(8)))
      for c in range(8):
        crow[pl.ds(c * 16, 16)] = tot[c]
      for c in range(8):
        colbuf[0, pl.ds(c * 16, 16)] = crow[pl.ds(c * 16, 16)]
      cidxbuf[pl.ds(0, 16)] = jnp.where(iota16 == 0, g_first, TRASH)
      pltpu.sync_copy(colbuf, ssum.at[cidxbuf], add=True)

    @pl.when(jnp.logical_not(g_uniform))
    def _():
      start_xscatter(p)
    pend_p = jnp.where(g_uniform, 0, 1)
    pend0, pend1 = (jnp.where(p == 0, pend_p, 0),
                    jnp.where(p == 1, pend_p, 0))

    def window_body(k, wcarry):
      cur_seg, fcount, valid = wcarry
      segv = segbuf[p, pl.ds(k * 16, 16)]
      yv = ybuf[p, pl.ds(k * 16, 16)]
      seg_first = segv[0]
      seg_last = segv[15]

      spill = fcount > GROUP - 17

      @pl.when(spill)
      def _():
        scatter_counts()
      fcount = jnp.where(spill, 0, fcount)

      new_run = seg_first != cur_seg
      do_flush = jnp.logical_and(new_run, valid == 1)

      @pl.when(do_flush)
      def _(cur_seg=cur_seg, fcount=fcount):
        flush(cur_seg, fcount)

      fcount = jnp.where(do_flush, fcount + 1, fcount)

      def fast(fc):
        ay = aycbuf[pl.ds(0, 16)]
        nay = jnp.where(new_run, 0.0, ay)
        hs = jnp.where(iota16 == yv[0], 1.0, 0.0).astype(jnp.float32)
        for j in range(1, 16):
          hs = hs + jnp.where(iota16 == yv[j], 1.0, 0.0).astype(jnp.float32)
        aycbuf[pl.ds(0, 16)] = nay + hs
        return fc

      def slow(fc):
        cur = seg_first
        for j in range(16):
          seg_r = segv[j]
          y_r = yv[j]
          is_new = seg_r != cur

          @pl.when(is_new)
          def _(cur=cur, fc=fc):
            flush(cur, fc)

          fc = jnp.where(is_new, fc + 1, fc)
          reset = jnp.logical_or(is_new, new_run) if j == 0 else is_new
          oh = jnp.where(iota16 == y_r, 1.0, 0.0).astype(jnp.float32)
          ay = aycbuf[pl.ds(0, 16)]
          aycbuf[pl.ds(0, 16)] = jnp.where(reset, oh, ay + oh)
          cur = seg_r
        return fc

      uniform = seg_first == seg_last
      fcount = lax.cond(uniform, fast, slow, fcount)
      return (seg_last, fcount, jnp.int32(1))

    cur_seg, fcount, valid = lax.fori_loop(0, GROUP // 16, window_body,
                                           (cur_seg, fcount, valid))
    return (cur_seg, fcount, valid, pend0, pend1)

  init = (jnp.int32(-1), jnp.int32(0), jnp.int32(0),
          jnp.int32(0), jnp.int32(0))
  cur_seg, fcount, valid, pend0, pend1 = lax.fori_loop(
      g_lo, g_hi, group_body, init)

  @pl.when(pend0 == 1)
  def _():
    wait_xscatter(0)

  @pl.when(pend1 == 1)
  def _():
    wait_xscatter(1)

  @pl.when(valid == 1)
  def _():
    flush(cur_seg, fcount)
  scatter_counts()

  plsc.subcore_barrier()
  base = sid * ROWS_PER_TILE
  pltpu.sync_copy(ssum.at[pl.ds(base, ROWS_PER_TILE)],
                  psum_hbm.at[cid, pl.ds(base, ROWS_PER_TILE)])
  pltpu.sync_copy(syc.at[pl.ds(base, ROWS_PER_TILE)],
                  pyc_hbm.at[cid, pl.ds(base, ROWS_PER_TILE)])


_sc_call = pl.kernel(
    _sc_body,
    out_type=(
        jax.ShapeDtypeStruct((NC, S, D), jnp.float32),
        jax.ShapeDtypeStruct((NC, S, D), jnp.float32),
    ),
    mesh=plsc.VectorSubcoreMesh(core_axis_name="c", subcore_axis_name="s"),
    scratch_types=[
        pltpu.VMEM((2, GROUP, D), jnp.float32),
        pltpu.VMEM((2, GROUP), jnp.int32),
        pltpu.VMEM((2, GROUP), jnp.int32),
        pltpu.VMEM((GROUP, D), jnp.float32),
        pltpu.VMEM((GROUP,), jnp.int32),
        pltpu.VMEM((16,), jnp.float32),
        pltpu.VMEM((16, D), jnp.float32),
        pltpu.VMEM((16,), jnp.int32),
        pltpu.VMEM((D,), jnp.float32),
        pltpu.VMEM_SHARED((SROWS, D), jnp.float32),
        pltpu.VMEM_SHARED((SROWS, D), jnp.float32),
        pltpu.SemaphoreType.DMA((2,)),
        pltpu.SemaphoreType.DMA((2,)),
    ],
)


def _fin_body(ps_ref, pyc_ref, xs_ref, ys_ref, m_ref):
  sums = ps_ref[0] + ps_ref[1]
  yc = pyc_ref[0] + pyc_ref[1]
  counts = jnp.sum(yc, axis=1)
  xs_ref[...] = sums / jnp.maximum(counts, 1.0)[:, None]
  mx = jnp.max(yc, axis=1, keepdims=True)
  lane = lax.broadcasted_iota(jnp.int32, (S, D), 1)
  idx = jnp.min(jnp.where(yc >= mx, lane, D), axis=1)
  ys_ref[...] = jnp.where(counts > 0, idx, -1)
  m_ref[...] = (counts > 0).astype(jnp.int32)


_fin_call = pl.pallas_call(
    _fin_body,
    out_shape=(
        jax.ShapeDtypeStruct((S, D), jnp.float32),
        jax.ShapeDtypeStruct((S,), jnp.int32),
        jax.ShapeDtypeStruct((S,), jnp.int32),
    ),
)


def kernel(x, segment_ids, y):
  seg2 = segment_ids.astype(jnp.int32).reshape(G, GROUP)
  y2 = y.astype(jnp.int32).reshape(G, GROUP)
  psum, pyc = _sc_call(x, seg2, y2)
  x_syn, y_syn, m = _fin_call(psum, pyc)
  return (x_syn, y_syn, m != 0)


# speedup vs baseline: 1.0243x; 1.0243x over previous
import jax
import jax.numpy as jnp
from jax import lax
from jax.experimental import pallas as pl
from jax.experimental.pallas import tpu as pltpu
from jax.experimental.pallas import tpu_sc as plsc

N = 320000
D = 128
S = 3200
C = 8
NC = 2
NS = 16
NW = NC * NS
GROUP = 128
G = N // GROUP
SROWS = 3328
TRASH = S
ZROWS_PER_TILE = SROWS // NS
ROWS_PER_TILE = S // NS


def _sc_body(x_hbm, seg_hbm, y_hbm, psum_hbm, pyc_hbm,
             xbuf, segbuf, ybuf, fybuf, fidxv, aycbuf,
             ssum, syc, lsem, ssem):
  cid = lax.axis_index("c")
  sid = lax.axis_index("s")
  wid = cid * NS + sid

  zeros16 = jnp.zeros((16,), jnp.float32)
  trash16 = jnp.full((16,), TRASH, jnp.int32)
  iota16 = lax.iota(jnp.int32, 16)

  def zrow(r, _):
    for c in range(D // 16):
      xbuf[0, r, pl.ds(c * 16, 16)] = zeros16
      fybuf[r, pl.ds(c * 16, 16)] = zeros16
    return 0
  lax.fori_loop(0, GROUP, zrow, 0)
  for b in range(8):
    fidxv[pl.ds(b * 16, 16)] = trash16

  zbase = sid * ZROWS_PER_TILE
  pltpu.sync_copy(xbuf.at[0, pl.ds(0, 128)], ssum.at[pl.ds(zbase, 128)])
  pltpu.sync_copy(xbuf.at[0, pl.ds(0, 80)], ssum.at[pl.ds(zbase + 128, 80)])
  pltpu.sync_copy(xbuf.at[0, pl.ds(0, 128)], syc.at[pl.ds(zbase, 128)])
  pltpu.sync_copy(xbuf.at[0, pl.ds(0, 80)], syc.at[pl.ds(zbase + 128, 80)])
  plsc.subcore_barrier()

  g_lo = wid * G // NW
  g_hi = (wid + 1) * G // NW

  def start_loads(g, p):
    pltpu.async_copy(x_hbm.at[pl.ds(g * GROUP, GROUP)], xbuf.at[p],
                     lsem.at[p])
    pltpu.async_copy(seg_hbm.at[g], segbuf.at[p], lsem.at[p])
    pltpu.async_copy(y_hbm.at[g], ybuf.at[p], lsem.at[p])

  def wait_loads(g, p):
    pltpu.make_async_copy(x_hbm.at[pl.ds(g * GROUP, GROUP)], xbuf.at[p],
                          lsem.at[p]).wait()
    pltpu.make_async_copy(seg_hbm.at[g], segbuf.at[p], lsem.at[p]).wait()
    pltpu.make_async_copy(y_hbm.at[g], ybuf.at[p], lsem.at[p]).wait()

  def start_xscatter(p):
    pltpu.async_copy(xbuf.at[p], ssum.at[segbuf.at[p]], ssem.at[p],
                     add=True)

  def wait_xscatter(p):
    pltpu.make_async_copy(xbuf.at[p], ssum.at[segbuf.at[p]],
                          ssem.at[p]).wait()

  def scatter_counts():
    pltpu.sync_copy(fybuf, syc.at[fidxv], add=True)
    for b in range(8):
      fidxv[pl.ds(b * 16, 16)] = trash16

  def flush(seg_id, fcount):
    fybuf[fcount, pl.ds(0, 16)] = aycbuf[pl.ds(0, 16)]
    blk = (fcount // 16) * 16
    off = fcount % 16
    v = fidxv[pl.ds(blk, 16)]
    fidxv[pl.ds(blk, 16)] = jnp.where(iota16 == off, seg_id, v)

  start_loads(g_lo, 0)

  def group_body(g, carry):
    cur_seg, fcount, valid = carry
    p = (g - g_lo) % 2

    @pl.when(g > g_lo)
    def _():
      wait_xscatter(1 - p)

    @pl.when(g + 1 < g_hi)
    def _():
      start_loads(g + 1, 1 - p)

    wait_loads(g, p)
    start_xscatter(p)

    def window_body(k, wcarry):
      cur_seg, fcount, valid = wcarry
      segv = segbuf[p, pl.ds(k * 16, 16)]
      yv = ybuf[p, pl.ds(k * 16, 16)]
      seg_first = segv[0]
      seg_last = segv[15]

      spill = fcount > GROUP - 17

      @pl.when(spill)
      def _():
        scatter_counts()
      fcount = jnp.where(spill, 0, fcount)

      new_run = seg_first != cur_seg
      do_flush = jnp.logical_and(new_run, valid == 1)

      @pl.when(do_flush)
      def _(cur_seg=cur_seg, fcount=fcount):
        flush(cur_seg, fcount)

      fcount = jnp.where(do_flush, fcount + 1, fcount)

      def fast(fc):
        ay = aycbuf[pl.ds(0, 16)]
        nay = jnp.where(new_run, 0.0, ay)
        hs = jnp.where(iota16 == yv[0], 1.0, 0.0).astype(jnp.float32)
        for j in range(1, 16):
          hs = hs + jnp.where(iota16 == yv[j], 1.0, 0.0).astype(jnp.float32)
        aycbuf[pl.ds(0, 16)] = nay + hs
        return fc

      def slow(fc):
        cur = seg_first
        for j in range(16):
          seg_r = segv[j]
          y_r = yv[j]
          is_new = seg_r != cur

          @pl.when(is_new)
          def _(cur=cur, fc=fc):
            flush(cur, fc)

          fc = jnp.where(is_new, fc + 1, fc)
          reset = jnp.logical_or(is_new, new_run) if j == 0 else is_new
          oh = jnp.where(iota16 == y_r, 1.0, 0.0).astype(jnp.float32)
          ay = aycbuf[pl.ds(0, 16)]
          aycbuf[pl.ds(0, 16)] = jnp.where(reset, oh, ay + oh)
          cur = seg_r
        return fc

      uniform = seg_first == seg_last
      fcount = lax.cond(uniform, fast, slow, fcount)
      return (seg_last, fcount, jnp.int32(1))

    return lax.fori_loop(0, GROUP // 16, window_body,
                         (cur_seg, fcount, valid))

  init = (jnp.int32(-1), jnp.int32(0), jnp.int32(0))
  cur_seg, fcount, valid = lax.fori_loop(g_lo, g_hi, group_body, init)

  last_p = (g_hi - 1 - g_lo) % 2
  wait_xscatter(last_p)

  @pl.when(valid == 1)
  def _():
    flush(cur_seg, fcount)
  scatter_counts()

  plsc.subcore_barrier()
  base = sid * ROWS_PER_TILE
  pltpu.sync_copy(ssum.at[pl.ds(base, ROWS_PER_TILE)],
                  psum_hbm.at[cid, pl.ds(base, ROWS_PER_TILE)])
  pltpu.sync_copy(syc.at[pl.ds(base, ROWS_PER_TILE)],
                  pyc_hbm.at[cid, pl.ds(base, ROWS_PER_TILE)])


_sc_call = pl.kernel(
    _sc_body,
    out_type=(
        jax.ShapeDtypeStruct((NC, S, D), jnp.float32),
        jax.ShapeDtypeStruct((NC, S, D), jnp.float32),
    ),
    mesh=plsc.VectorSubcoreMesh(core_axis_name="c", subcore_axis_name="s"),
    scratch_types=[
        pltpu.VMEM((2, GROUP, D), jnp.float32),
        pltpu.VMEM((2, GROUP), jnp.int32),
        pltpu.VMEM((2, GROUP), jnp.int32),
        pltpu.VMEM((GROUP, D), jnp.float32),
        pltpu.VMEM((GROUP,), jnp.int32),
        pltpu.VMEM((16,), jnp.float32),
        pltpu.VMEM_SHARED((SROWS, D), jnp.float32),
        pltpu.VMEM_SHARED((SROWS, D), jnp.float32),
        pltpu.SemaphoreType.DMA((2,)),
        pltpu.SemaphoreType.DMA((2,)),
    ],
)


def _fin_body(ps_ref, pyc_ref, xs_ref, ys_ref, m_ref):
  sums = ps_ref[0] + ps_ref[1]
  yc = pyc_ref[0] + pyc_ref[1]
  counts = jnp.sum(yc, axis=1)
  xs_ref[...] = sums / jnp.maximum(counts, 1.0)[:, None]
  mx = jnp.max(yc, axis=1, keepdims=True)
  lane = lax.broadcasted_iota(jnp.int32, (S, D), 1)
  idx = jnp.min(jnp.where(yc >= mx, lane, D), axis=1)
  ys_ref[...] = jnp.where(counts > 0, idx, -1)
  m_ref[...] = (counts > 0).astype(jnp.int32)


_fin_call = pl.pallas_call(
    _fin_body,
    out_shape=(
        jax.ShapeDtypeStruct((S, D), jnp.float32),
        jax.ShapeDtypeStruct((S,), jnp.int32),
        jax.ShapeDtypeStruct((S,), jnp.int32),
    ),
)


def kernel(x, segment_ids, y):
  seg2 = segment_ids.astype(jnp.int32).reshape(G, GROUP)
  y2 = y.astype(jnp.int32).reshape(G, GROUP)
  psum, pyc = _sc_call(x, seg2, y2)
  x_syn, y_syn, m = _fin_call(psum, pyc)
  return (x_syn, y_syn, m != 0)
